# baseline (speedup 1.0000x reference)
import jax
import jax.numpy as jnp
from jax.experimental import pallas as pl
from jax.experimental.pallas import tpu as pltpu

B, T, C, H, W = 32, 256, 3, 32, 32
CHW = C * H * W
FEAT = 512
HID = 128
TOP_K = 8
G3 = 3 * HID
TT = 8
NT = T // TT
UN = 8


def _dot_nt(a, b):
    return jax.lax.dot_general(a, b, (((1,), (1,)), ((), ())),
                               preferred_element_type=jnp.float32)


def _fused_kernel(x_ref, wenc_ref, benc_ref, wih_ref, bih_ref,
                  whhf_ref, whhb_ref, bhhf_ref, bhhb_ref,
                  lenc_ref, lenr_ref, temp_ref, b2_ref,
                  w1_ref, b1_ref, w2r_ref, whead_ref, bhead_ref,
                  out_ref, gi_s, outs_ref, hf_s, hb_s):
    i = pl.program_id(0)
    lenc = lenc_ref[...]

    @pl.when(i < NT)
    def _enc():
        x = x_ref[...].reshape(B * TT, CHW)
        f = _dot_nt(x, wenc_ref[...]) + benc_ref[...]
        g = _dot_nt(f, wih_ref[...]) + bih_ref[...]
        gt = jnp.transpose(g.reshape(B, TT, 2 * G3), (1, 0, 2))
        gi_s[pl.ds(i * TT * B, TT * B)] = gt.reshape(TT * B, 2 * G3)

    @pl.when(i == 0)
    def _init():
        hf_s[...] = jnp.zeros((B, HID), jnp.float32)
        hb_s[...] = jnp.zeros((B, HID), jnp.float32)

    def cell(gi_t, h, whh, bhh):
        gh = jnp.dot(h.astype(jnp.bfloat16), whh,
                     preferred_element_type=jnp.float32) + bhh
        r = jax.nn.sigmoid(gi_t[:, :HID] + gh[:, :HID])
        z = jax.nn.sigmoid(gi_t[:, HID:2 * HID] + gh[:, HID:2 * HID])
        n = jnp.tanh(gi_t[:, 2 * HID:] + r * gh[:, 2 * HID:])
        return (1.0 - z) * n + z * h

    @pl.when(i > 0)
    def _fwd():
        whhf = whhf_ref[...]
        bhhf = bhhf_ref[...]
        hf = hf_s[...]
        base = (i - 1) * TT
        for j in range(TT):
            t = base + j
            gif = gi_s[pl.ds(t * B, B), :G3]
            hfn = cell(gif, hf, whhf, bhhf)
            vf = t < lenc
            outs_ref[pl.ds(t * B, B), :HID] = jnp.where(vf, hfn, 0.0)
            hf = jnp.where(vf, hfn, hf)
        hf_s[...] = hf

    @pl.when(i == NT)
    def _bwd_and_attend():
        whhb = whhb_ref[...]
        bhhb = bhhb_ref[...]

        def bstep(k, hb):
            for j in range(UN):
                t = T - 1 - (UN * k + j)
                gib = gi_s[pl.ds(t * B, B), G3:]
                hbn = cell(gib, hb, whhb, bhhb)
                vb = t < lenc
                outs_ref[pl.ds(t * B, B), HID:] = jnp.where(vb, hbn, 0.0)
                hb = jnp.where(vb, hbn, hb)
            return hb

        jax.lax.fori_loop(0, T // UN, bstep,
                          jnp.zeros((B, HID), jnp.float32))

        flat = outs_ref[...]
        h1 = jnp.maximum(_dot_nt(flat, w1_ref[...]) + b1_ref[...], 0.0)
        m = jnp.dot(h1, w2r_ref[...],
                    preferred_element_type=jnp.float32)

        ridx = jax.lax.broadcasted_iota(jnp.int32, (T * B, B), 0)
        lidx = jax.lax.broadcasted_iota(jnp.int32, (T * B, B), 1)
        eyer = (ridx % B) == lidx
        msel = jnp.where(eyer, m, 0.0)
        scores = jnp.sum(msel.reshape(T, B, B), axis=1) + b2_ref[...]

        lenr = lenr_ref[...]
        tio = jax.lax.broadcasted_iota(jnp.int32, (T, B), 0)
        valid = tio < lenr
        temp = jnp.clip(temp_ref[...], 0.001, 10.0)
        logits = jnp.where(valid, scores, -jnp.inf) / temp
        mx = jnp.max(logits, axis=0, keepdims=True)
        e = jnp.exp(logits - mx)
        probs = e / jnp.sum(e, axis=0, keepdims=True)

        kact = jnp.minimum(lenr, TOP_K)
        work = probs
        acc = jnp.zeros((T, B), jnp.float32)
        vsum = jnp.zeros((1, B), jnp.float32)
        for k in range(TOP_K):
            v = jnp.max(work, axis=0, keepdims=True)
            hit = work == v
            idx = jnp.min(jnp.where(hit, tio, T), axis=0, keepdims=True)
            onehot = tio == idx
            ind = (k < kact).astype(jnp.float32)
            acc = acc + jnp.where(onehot, v * ind, 0.0)
            vsum = vsum + v * ind
            work = jnp.where(onehot, -1.0, work)
        att = acc / jnp.maximum(vsum, 1e-12)
        maskf = valid.astype(jnp.float32)
        uni = maskf / (jnp.sum(maskf, axis=0, keepdims=True) + 1e-8)
        att = jnp.where(vsum > 1e-8, att, uni)

        attr = jnp.broadcast_to(att[:, None, :], (T, B, B)).reshape(T * B, B)
        attc = jnp.sum(jnp.where(eyer, attr, 0.0), axis=1, keepdims=True)
        seq = jnp.sum((flat * attc).reshape(T, B, 2 * HID), axis=0)
        out_ref[...] = _dot_nt(seq, whead_ref[...]) + bhead_ref[...]


def kernel(frames, params, lengths):
    p = params
    x = frames.reshape(B, T, CHW)
    wih = jnp.concatenate([p['gru_fwd']['W_ih'], p['gru_bwd']['W_ih']], axis=0)
    bih = jnp.concatenate([p['gru_fwd']['b_ih'], p['gru_bwd']['b_ih']]).reshape(1, 2 * G3)
    benc = p['b_enc'].reshape(1, FEAT)
    bhhf = p['gru_fwd']['b_hh'].reshape(1, G3)
    bhhb = p['gru_bwd']['b_hh'].reshape(1, G3)
    lenc = lengths.reshape(B, 1)
    lenr = lengths.reshape(1, B)
    tempc = p['temperature'].reshape(1, 1)
    b1 = p['b1'].reshape(1, 64)
    w2r = jnp.broadcast_to(p['W2'].reshape(64, 1), (64, B))
    b2 = p['b2'].reshape(1, 1)
    whead = jnp.concatenate([p['Wt'], p['Wo']], axis=0)
    bhead = jnp.concatenate([p['bt'], p['bo']]).reshape(1, 21)

    const = lambda i: (0, 0)
    heads = pl.pallas_call(
        _fused_kernel,
        grid=(NT + 1,),
        in_specs=[
            pl.BlockSpec((B, TT, CHW), lambda i: (0, jnp.minimum(i, NT - 1), 0)),
            pl.BlockSpec((FEAT, CHW), const),
            pl.BlockSpec((1, FEAT), const),
            pl.BlockSpec((2 * G3, FEAT), const),
            pl.BlockSpec((1, 2 * G3), const),
            pl.BlockSpec((HID, G3), const),
            pl.BlockSpec((HID, G3), const),
            pl.BlockSpec((1, G3), const),
            pl.BlockSpec((1, G3), const),
            pl.BlockSpec((B, 1), const),
            pl.BlockSpec((1, B), const),
            pl.BlockSpec((1, 1), const),
            pl.BlockSpec((1, 1), const),
            pl.BlockSpec((64, 2 * HID), const),
            pl.BlockSpec((1, 64), const),
            pl.BlockSpec((64, B), const),
            pl.BlockSpec((21, 2 * HID), const),
            pl.BlockSpec((1, 21), const),
        ],
        out_specs=pl.BlockSpec((B, 21), const),
        out_shape=jax.ShapeDtypeStruct((B, 21), jnp.float32),
        scratch_shapes=[
            pltpu.VMEM((T * B, 2 * G3), jnp.float32),
            pltpu.VMEM((T * B, 2 * HID), jnp.float32),
            pltpu.VMEM((B, HID), jnp.float32),
            pltpu.VMEM((B, HID), jnp.float32),
        ],
    )(x, p['W_enc'], benc, wih, bih,
      p['gru_fwd']['W_hh'].T.astype(jnp.bfloat16),
      p['gru_bwd']['W_hh'].T.astype(jnp.bfloat16),
      bhhf, bhhb, lenc, lenr, tempc, b2,
      p['W1'], b1, w2r, whead, bhead)

    return heads[:, :11], heads[:, 11:21]


# speedup vs baseline: 1.0330x; 1.0330x over previous
import jax
import jax.numpy as jnp
from jax.experimental import pallas as pl
from jax.experimental.pallas import tpu as pltpu

B, T, C, H, W = 32, 256, 3, 32, 32
CHW = C * H * W
FEAT = 512
HID = 128
TOP_K = 8
G3 = 3 * HID
TT = 16
NT = T // TT
UN = 8


def _dot_nt(a, b):
    return jax.lax.dot_general(a, b, (((1,), (1,)), ((), ())),
                               preferred_element_type=jnp.float32)


def _fused_kernel(x_ref, wenc_ref, benc_ref, wih_ref, bih_ref,
                  whhf_ref, whhb_ref, bhhf_ref, bhhb_ref,
                  lenc_ref, lenr_ref, temp_ref, b2_ref,
                  w1_ref, b1_ref, w2r_ref, whead_ref, bhead_ref,
                  out_ref, gi_s, outs_ref, hf_s, hb_s):
    i = pl.program_id(0)
    lenc = lenc_ref[...]

    @pl.when(i < NT)
    def _enc():
        x = x_ref[...].reshape(B * TT, CHW)
        f = _dot_nt(x, wenc_ref[...]) + benc_ref[...]
        g = _dot_nt(f, wih_ref[...]) + bih_ref[...]
        gt = jnp.transpose(g.reshape(B, TT, 2 * G3), (1, 0, 2))
        gi_s[pl.ds(i * TT * B, TT * B)] = gt.reshape(TT * B, 2 * G3)

    @pl.when(i == 0)
    def _init():
        hf_s[...] = jnp.zeros((B, HID), jnp.float32)
        hb_s[...] = jnp.zeros((B, HID), jnp.float32)

    def cell(gi_t, h, whh, bhh):
        gh = jnp.dot(h.astype(jnp.bfloat16), whh,
                     preferred_element_type=jnp.float32) + bhh
        r = jax.nn.sigmoid(gi_t[:, :HID] + gh[:, :HID])
        z = jax.nn.sigmoid(gi_t[:, HID:2 * HID] + gh[:, HID:2 * HID])
        n = jnp.tanh(gi_t[:, 2 * HID:] + r * gh[:, 2 * HID:])
        return (1.0 - z) * n + z * h

    @pl.when(i > 0)
    def _fwd():
        whhf = whhf_ref[...]
        bhhf = bhhf_ref[...]
        hf = hf_s[...]
        base = (i - 1) * TT
        for j in range(TT):
            t = base + j
            gif = gi_s[pl.ds(t * B, B), :G3]
            hfn = cell(gif, hf, whhf, bhhf)
            vf = t < lenc
            outs_ref[pl.ds(t * B, B), :HID] = jnp.where(vf, hfn, 0.0)
            hf = jnp.where(vf, hfn, hf)
        hf_s[...] = hf

    @pl.when(i == NT)
    def _bwd_and_attend():
        whhb = whhb_ref[...]
        bhhb = bhhb_ref[...]

        def bstep(k, hb):
            for j in range(UN):
                t = T - 1 - (UN * k + j)
                gib = gi_s[pl.ds(t * B, B), G3:]
                hbn = cell(gib, hb, whhb, bhhb)
                vb = t < lenc
                outs_ref[pl.ds(t * B, B), HID:] = jnp.where(vb, hbn, 0.0)
                hb = jnp.where(vb, hbn, hb)
            return hb

        jax.lax.fori_loop(0, T // UN, bstep,
                          jnp.zeros((B, HID), jnp.float32))

        flat = outs_ref[...]
        h1 = jnp.maximum(_dot_nt(flat, w1_ref[...]) + b1_ref[...], 0.0)
        m = jnp.dot(h1, w2r_ref[...],
                    preferred_element_type=jnp.float32)

        ridx = jax.lax.broadcasted_iota(jnp.int32, (T * B, B), 0)
        lidx = jax.lax.broadcasted_iota(jnp.int32, (T * B, B), 1)
        eyer = (ridx % B) == lidx
        msel = jnp.where(eyer, m, 0.0)
        scores = jnp.sum(msel.reshape(T, B, B), axis=1) + b2_ref[...]

        lenr = lenr_ref[...]
        tio = jax.lax.broadcasted_iota(jnp.int32, (T, B), 0)
        valid = tio < lenr
        temp = jnp.clip(temp_ref[...], 0.001, 10.0)
        logits = jnp.where(valid, scores, -jnp.inf) / temp
        mx = jnp.max(logits, axis=0, keepdims=True)
        e = jnp.exp(logits - mx)
        probs = e / jnp.sum(e, axis=0, keepdims=True)

        kact = jnp.minimum(lenr, TOP_K)
        work = probs
        acc = jnp.zeros((T, B), jnp.float32)
        vsum = jnp.zeros((1, B), jnp.float32)
        for k in range(TOP_K):
            v = jnp.max(work, axis=0, keepdims=True)
            hit = work == v
            idx = jnp.min(jnp.where(hit, tio, T), axis=0, keepdims=True)
            onehot = tio == idx
            ind = (k < kact).astype(jnp.float32)
            acc = acc + jnp.where(onehot, v * ind, 0.0)
            vsum = vsum + v * ind
            work = jnp.where(onehot, -1.0, work)
        att = acc / jnp.maximum(vsum, 1e-12)
        maskf = valid.astype(jnp.float32)
        uni = maskf / (jnp.sum(maskf, axis=0, keepdims=True) + 1e-8)
        att = jnp.where(vsum > 1e-8, att, uni)

        attr = jnp.broadcast_to(att[:, None, :], (T, B, B)).reshape(T * B, B)
        attc = jnp.sum(jnp.where(eyer, attr, 0.0), axis=1, keepdims=True)
        seq = jnp.sum((flat * attc).reshape(T, B, 2 * HID), axis=0)
        out_ref[...] = _dot_nt(seq, whead_ref[...]) + bhead_ref[...]


def kernel(frames, params, lengths):
    p = params
    x = frames.reshape(B, T, CHW)
    wih = jnp.concatenate([p['gru_fwd']['W_ih'], p['gru_bwd']['W_ih']], axis=0)
    bih = jnp.concatenate([p['gru_fwd']['b_ih'], p['gru_bwd']['b_ih']]).reshape(1, 2 * G3)
    benc = p['b_enc'].reshape(1, FEAT)
    bhhf = p['gru_fwd']['b_hh'].reshape(1, G3)
    bhhb = p['gru_bwd']['b_hh'].reshape(1, G3)
    lenc = lengths.reshape(B, 1)
    lenr = lengths.reshape(1, B)
    tempc = p['temperature'].reshape(1, 1)
    b1 = p['b1'].reshape(1, 64)
    w2r = jnp.broadcast_to(p['W2'].reshape(64, 1), (64, B))
    b2 = p['b2'].reshape(1, 1)
    whead = jnp.concatenate([p['Wt'], p['Wo']], axis=0)
    bhead = jnp.concatenate([p['bt'], p['bo']]).reshape(1, 21)

    const = lambda i: (0, 0)
    heads = pl.pallas_call(
        _fused_kernel,
        grid=(NT + 1,),
        in_specs=[
            pl.BlockSpec((B, TT, CHW), lambda i: (0, jnp.minimum(i, NT - 1), 0)),
            pl.BlockSpec((FEAT, CHW), const),
            pl.BlockSpec((1, FEAT), const),
            pl.BlockSpec((2 * G3, FEAT), const),
            pl.BlockSpec((1, 2 * G3), const),
            pl.BlockSpec((HID, G3), const),
            pl.BlockSpec((HID, G3), const),
            pl.BlockSpec((1, G3), const),
            pl.BlockSpec((1, G3), const),
            pl.BlockSpec((B, 1), const),
            pl.BlockSpec((1, B), const),
            pl.BlockSpec((1, 1), const),
            pl.BlockSpec((1, 1), const),
            pl.BlockSpec((64, 2 * HID), const),
            pl.BlockSpec((1, 64), const),
            pl.BlockSpec((64, B), const),
            pl.BlockSpec((21, 2 * HID), const),
            pl.BlockSpec((1, 21), const),
        ],
        out_specs=pl.BlockSpec((B, 21), const),
        out_shape=jax.ShapeDtypeStruct((B, 21), jnp.float32),
        scratch_shapes=[
            pltpu.VMEM((T * B, 2 * G3), jnp.float32),
            pltpu.VMEM((T * B, 2 * HID), jnp.float32),
            pltpu.VMEM((B, HID), jnp.float32),
            pltpu.VMEM((B, HID), jnp.float32),
        ],
    )(x, p['W_enc'], benc, wih, bih,
      p['gru_fwd']['W_hh'].T.astype(jnp.bfloat16),
      p['gru_bwd']['W_hh'].T.astype(jnp.bfloat16),
      bhhf, bhhb, lenc, lenr, tempc, b2,
      p['W1'], b1, w2r, whead, bhead)

    return heads[:, :11], heads[:, 11:21]


# speedup vs baseline: 1.0344x; 1.0014x over previous
import jax
import jax.numpy as jnp
from jax.experimental import pallas as pl
from jax.experimental.pallas import tpu as pltpu

B, T, C, H, W = 32, 256, 3, 32, 32
CHW = C * H * W
FEAT = 512
HID = 128
TOP_K = 8
G3 = 3 * HID
TT = 16
NT = T // TT
UN = 8


def _dot_nt(a, b):
    return jax.lax.dot_general(a, b, (((1,), (1,)), ((), ())),
                               preferred_element_type=jnp.float32)


def _fused_kernel(x_ref, wenc_ref, benc_ref, wih_ref, bih_ref,
                  whhf_ref, whhb_ref, bhhf_ref, bhhb_ref,
                  lenc_ref, lenr_ref, temp_ref, b2_ref,
                  w1_ref, b1_ref, w2r_ref, whead_ref, bhead_ref,
                  out_ref, gi_s, outs_ref, hf_s, hb_s):
    i = pl.program_id(0)
    lenc = lenc_ref[...]

    @pl.when(i < NT)
    def _enc():
        x = x_ref[...].reshape(B * TT, CHW)
        f = _dot_nt(x, wenc_ref[...]) + benc_ref[...]
        g = _dot_nt(f, wih_ref[...]) + bih_ref[...]
        gt = jnp.transpose(g.reshape(B, TT, 2 * G3), (1, 0, 2))
        gi_s[pl.ds(i * TT * B, TT * B)] = (
            gt.reshape(TT * B, 2 * G3).astype(jnp.bfloat16))

    @pl.when(i == 0)
    def _init():
        hf_s[...] = jnp.zeros((B, HID), jnp.float32)
        hb_s[...] = jnp.zeros((B, HID), jnp.float32)

    def cell(gi_t, h, whh, bhh):
        gh = jnp.dot(h.astype(jnp.bfloat16), whh,
                     preferred_element_type=jnp.float32) + bhh
        r = jax.nn.sigmoid(gi_t[:, :HID] + gh[:, :HID])
        z = jax.nn.sigmoid(gi_t[:, HID:2 * HID] + gh[:, HID:2 * HID])
        n = jnp.tanh(gi_t[:, 2 * HID:] + r * gh[:, 2 * HID:])
        return (1.0 - z) * n + z * h

    @pl.when(i > 0)
    def _fwd():
        whhf = whhf_ref[...]
        bhhf = bhhf_ref[...]
        hf = hf_s[...]
        base = (i - 1) * TT
        for j in range(TT):
            t = base + j
            gif = gi_s[pl.ds(t * B, B), :G3].astype(jnp.float32)
            hfn = cell(gif, hf, whhf, bhhf)
            vf = t < lenc
            outs_ref[pl.ds(t * B, B), :HID] = jnp.where(vf, hfn, 0.0)
            hf = jnp.where(vf, hfn, hf)
        hf_s[...] = hf

    @pl.when(i == NT)
    def _bwd_and_attend():
        whhb = whhb_ref[...]
        bhhb = bhhb_ref[...]

        def bstep(k, hb):
            for j in range(UN):
                t = T - 1 - (UN * k + j)
                gib = gi_s[pl.ds(t * B, B), G3:].astype(jnp.float32)
                hbn = cell(gib, hb, whhb, bhhb)
                vb = t < lenc
                outs_ref[pl.ds(t * B, B), HID:] = jnp.where(vb, hbn, 0.0)
                hb = jnp.where(vb, hbn, hb)
            return hb

        jax.lax.fori_loop(0, T // UN, bstep,
                          jnp.zeros((B, HID), jnp.float32))

        flat = outs_ref[...]
        h1 = jnp.maximum(_dot_nt(flat, w1_ref[...]) + b1_ref[...], 0.0)
        m = jnp.dot(h1, w2r_ref[...],
                    preferred_element_type=jnp.float32)

        ridx = jax.lax.broadcasted_iota(jnp.int32, (T * B, B), 0)
        lidx = jax.lax.broadcasted_iota(jnp.int32, (T * B, B), 1)
        eyer = (ridx % B) == lidx
        msel = jnp.where(eyer, m, 0.0)
        scores = jnp.sum(msel.reshape(T, B, B), axis=1) + b2_ref[...]

        lenr = lenr_ref[...]
        tio = jax.lax.broadcasted_iota(jnp.int32, (T, B), 0)
        valid = tio < lenr
        temp = jnp.clip(temp_ref[...], 0.001, 10.0)
        logits = jnp.where(valid, scores, -jnp.inf) / temp
        mx = jnp.max(logits, axis=0, keepdims=True)
        e = jnp.exp(logits - mx)
        probs = e / jnp.sum(e, axis=0, keepdims=True)

        kact = jnp.minimum(lenr, TOP_K)
        work = probs
        acc = jnp.zeros((T, B), jnp.float32)
        vsum = jnp.zeros((1, B), jnp.float32)
        for k in range(TOP_K):
            v = jnp.max(work, axis=0, keepdims=True)
            hit = work == v
            idx = jnp.min(jnp.where(hit, tio, T), axis=0, keepdims=True)
            onehot = tio == idx
            ind = (k < kact).astype(jnp.float32)
            acc = acc + jnp.where(onehot, v * ind, 0.0)
            vsum = vsum + v * ind
            work = jnp.where(onehot, -1.0, work)
        att = acc / jnp.maximum(vsum, 1e-12)
        maskf = valid.astype(jnp.float32)
        uni = maskf / (jnp.sum(maskf, axis=0, keepdims=True) + 1e-8)
        att = jnp.where(vsum > 1e-8, att, uni)

        attr = jnp.broadcast_to(att[:, None, :], (T, B, B)).reshape(T * B, B)
        attc = jnp.sum(jnp.where(eyer, attr, 0.0), axis=1, keepdims=True)
        seq = jnp.sum((flat * attc).reshape(T, B, 2 * HID), axis=0)
        out_ref[...] = _dot_nt(seq, whead_ref[...]) + bhead_ref[...]


def kernel(frames, params, lengths):
    p = params
    x = frames.reshape(B, T, CHW)
    wih = jnp.concatenate([p['gru_fwd']['W_ih'], p['gru_bwd']['W_ih']], axis=0)
    bih = jnp.concatenate([p['gru_fwd']['b_ih'], p['gru_bwd']['b_ih']]).reshape(1, 2 * G3)
    benc = p['b_enc'].reshape(1, FEAT)
    bhhf = p['gru_fwd']['b_hh'].reshape(1, G3)
    bhhb = p['gru_bwd']['b_hh'].reshape(1, G3)
    lenc = lengths.reshape(B, 1)
    lenr = lengths.reshape(1, B)
    tempc = p['temperature'].reshape(1, 1)
    b1 = p['b1'].reshape(1, 64)
    w2r = jnp.broadcast_to(p['W2'].reshape(64, 1), (64, B))
    b2 = p['b2'].reshape(1, 1)
    whead = jnp.concatenate([p['Wt'], p['Wo']], axis=0)
    bhead = jnp.concatenate([p['bt'], p['bo']]).reshape(1, 21)

    const = lambda i: (0, 0)
    heads = pl.pallas_call(
        _fused_kernel,
        grid=(NT + 1,),
        in_specs=[
            pl.BlockSpec((B, TT, CHW), lambda i: (0, jnp.minimum(i, NT - 1), 0)),
            pl.BlockSpec((FEAT, CHW), const),
            pl.BlockSpec((1, FEAT), const),
            pl.BlockSpec((2 * G3, FEAT), const),
            pl.BlockSpec((1, 2 * G3), const),
            pl.BlockSpec((HID, G3), const),
            pl.BlockSpec((HID, G3), const),
            pl.BlockSpec((1, G3), const),
            pl.BlockSpec((1, G3), const),
            pl.BlockSpec((B, 1), const),
            pl.BlockSpec((1, B), const),
            pl.BlockSpec((1, 1), const),
            pl.BlockSpec((1, 1), const),
            pl.BlockSpec((64, 2 * HID), const),
            pl.BlockSpec((1, 64), const),
            pl.BlockSpec((64, B), const),
            pl.BlockSpec((21, 2 * HID), const),
            pl.BlockSpec((1, 21), const),
        ],
        out_specs=pl.BlockSpec((B, 21), const),
        out_shape=jax.ShapeDtypeStruct((B, 21), jnp.float32),
        scratch_shapes=[
            pltpu.VMEM((T * B, 2 * G3), jnp.bfloat16),
            pltpu.VMEM((T * B, 2 * HID), jnp.float32),
            pltpu.VMEM((B, HID), jnp.float32),
            pltpu.VMEM((B, HID), jnp.float32),
        ],
    )(x, p['W_enc'], benc, wih, bih,
      p['gru_fwd']['W_hh'].T.astype(jnp.bfloat16),
      p['gru_bwd']['W_hh'].T.astype(jnp.bfloat16),
      bhhf, bhhb, lenc, lenr, tempc, b2,
      p['W1'], b1, w2r, whead, bhead)

    return heads[:, :11], heads[:, 11:21]
